# Initial kernel scaffold; baseline (speedup 1.0000x reference)
import jax
import jax.numpy as jnp
from jax.experimental import pallas as pl


def kernel(q, k, v, W):
    raise NotImplementedError("write your pallas kernel here")


# speedup vs baseline: 35.6458x; 35.6458x over previous
import functools
import math

import jax
import jax.numpy as jnp
from jax import lax
from jax.experimental import pallas as pl
from jax.experimental.pallas import tpu as pltpu
from jax.experimental.pallas import tpu_sc as plsc

NUM_BUCKETS = 32
MAX_DISTANCE = 128
N_HEADS = 16
SEQ = 2048
SHIFTS = 8
TBL = 4096
INFLIGHT = 16


def _table_body(w_ref, t_ref):
    h = pl.program_id(0)
    flat = lax.broadcasted_iota(jnp.int32, (1, 1, SHIFTS * TBL), 2)
    x = lax.bitwise_and(flat, TBL - 1)
    r = lax.shift_right_logical(flat, 12)
    d = x + r - (SEQ - 1)

    num_buckets = NUM_BUCKETS // 2
    max_exact = num_buckets // 2
    rel_buckets = jnp.where(d > 0, num_buckets, 0)
    rp = jnp.abs(d)
    is_small = rp < max_exact
    rp_safe = jnp.maximum(rp, 1).astype(jnp.float32)
    large = max_exact + (
        jnp.log(rp_safe / max_exact)
        / math.log(MAX_DISTANCE / max_exact)
        * (num_buckets - max_exact)
    ).astype(jnp.int32)
    large = jnp.minimum(large, num_buckets - 1)
    bucket = rel_buckets + jnp.where(is_small, rp, large)

    vals = jnp.zeros((1, 1, SHIFTS * TBL), jnp.float32)
    for b in range(NUM_BUCKETS):
        vals = jnp.where(bucket == b, w_ref[b, h], vals)
    t_ref[...] = vals


_build_table = pl.pallas_call(
    _table_body,
    grid=(N_HEADS,),
    in_specs=[pl.BlockSpec(memory_space=pltpu.SMEM)],
    out_specs=pl.BlockSpec((1, 1, SHIFTS * TBL), lambda h: (h, 0, 0)),
    out_shape=jax.ShapeDtypeStruct((N_HEADS, 1, SHIFTS * TBL), jnp.float32),
)


_sc_mesh = plsc.VectorSubcoreMesh(core_axis_name="c", subcore_axis_name="s")


@functools.partial(
    pl.kernel,
    out_type=jax.ShapeDtypeStruct((N_HEADS, SEQ, SEQ), jnp.float32),
    mesh=_sc_mesh,
    scratch_types=[
        pltpu.VMEM((SHIFTS * TBL,), jnp.float32),
        pltpu.SemaphoreType.DMA,
    ],
    compiler_params=pltpu.CompilerParams(use_tc_tiling_on_sc=False),
)
def _expand(t_hbm, out_hbm, tbl_v, sem):
    core = lax.axis_index("c")
    sub = lax.axis_index("s")
    h = sub
    base = core * (SEQ // 2)
    rows = SEQ // 2

    pltpu.sync_copy(t_hbm.at[h, 0], tbl_v)

    def row_copy(row):
        off = (SEQ - 1) - row
        r = lax.bitwise_and(off, SHIFTS - 1)
        cs = off - r
        src = pl.multiple_of(r * TBL + cs, SHIFTS)
        return pltpu.make_async_copy(
            tbl_v.at[pl.ds(src, SEQ)], out_hbm.at[h, row], sem
        )

    def body(i, _):
        row_copy(base + i).start()

        @pl.when(i >= INFLIGHT)
        def _():
            row_copy(base).wait()

        return 0

    lax.fori_loop(0, rows, body, 0)

    def drain(i, _):
        row_copy(base).wait()
        return 0

    lax.fori_loop(0, INFLIGHT, drain, 0)


@jax.jit
def _bias(w):
    table = _build_table(w)
    out = _expand(table)
    return out.reshape(1, N_HEADS, SEQ, SEQ)


def kernel(q, k, v, W):
    return (q, k, v, _bias(W))


# speedup vs baseline: 39.4153x; 1.1058x over previous
import functools
import math

import jax
import jax.numpy as jnp
from jax import lax
from jax.experimental import pallas as pl
from jax.experimental.pallas import tpu as pltpu
from jax.experimental.pallas import tpu_sc as plsc

NUM_BUCKETS = 32
MAX_DISTANCE = 128
N_HEADS = 16
SEQ = 2048
SHIFTS = 8
TBL = 4096
INFLIGHT = 16


def _table_body(w_ref, t_ref):
    flat = lax.broadcasted_iota(jnp.int32, (1, SHIFTS * TBL), 1)
    x = lax.bitwise_and(flat, TBL - 1)
    r = lax.shift_right_logical(flat, 12)
    d = x + r - (SEQ - 1)

    num_buckets = NUM_BUCKETS // 2
    max_exact = num_buckets // 2
    rel_buckets = jnp.where(d > 0, num_buckets, 0)
    rp = jnp.abs(d)
    is_small = rp < max_exact
    rp_safe = jnp.maximum(rp, 1).astype(jnp.float32)
    large = max_exact + (
        jnp.log(rp_safe / max_exact)
        / math.log(MAX_DISTANCE / max_exact)
        * (num_buckets - max_exact)
    ).astype(jnp.int32)
    large = jnp.minimum(large, num_buckets - 1)
    bucket = rel_buckets + jnp.where(is_small, rp, large)

    bidx = lax.broadcasted_iota(jnp.int32, (NUM_BUCKETS, SHIFTS * TBL), 0)
    onehot = (bidx == bucket).astype(jnp.float32)
    t_ref[...] = lax.dot_general(
        w_ref[...], onehot, (((0,), (0,)), ((), ())),
        preferred_element_type=jnp.float32,
    )


_build_table = pl.pallas_call(
    _table_body,
    out_shape=jax.ShapeDtypeStruct((N_HEADS, SHIFTS * TBL), jnp.float32),
)


_sc_mesh = plsc.VectorSubcoreMesh(core_axis_name="c", subcore_axis_name="s")


@functools.partial(
    pl.kernel,
    out_type=jax.ShapeDtypeStruct((1, N_HEADS, SEQ, SEQ), jnp.float32),
    mesh=_sc_mesh,
    scratch_types=[
        pltpu.VMEM((SHIFTS * TBL,), jnp.float32),
        pltpu.SemaphoreType.DMA,
    ],
    compiler_params=pltpu.CompilerParams(use_tc_tiling_on_sc=False),
)
def _expand(t_hbm, out_hbm, tbl_v, sem):
    core = lax.axis_index("c")
    sub = lax.axis_index("s")
    h = sub
    base = core * (SEQ // 2)
    rows = SEQ // 2

    pltpu.sync_copy(t_hbm.at[h], tbl_v)

    def row_copy(row):
        off = (SEQ - 1) - row
        r = lax.bitwise_and(off, SHIFTS - 1)
        cs = off - r
        src = pl.multiple_of(r * TBL + cs, SHIFTS)
        return pltpu.make_async_copy(
            tbl_v.at[pl.ds(src, SEQ)], out_hbm.at[0, h, row], sem
        )

    def body(i, _):
        row_copy(base + i).start()

        @pl.when(i >= INFLIGHT)
        def _():
            row_copy(base).wait()

        return 0

    lax.fori_loop(0, rows, body, 0)

    def drain(i, _):
        row_copy(base).wait()
        return 0

    lax.fori_loop(0, INFLIGHT, drain, 0)


@jax.jit
def _bias(w):
    table = _build_table(w)
    return _expand(table)


def kernel(q, k, v, W):
    return (q, k, v, _bias(W))
